# baseline (speedup 1.0000x reference)
import functools

import jax
import jax.numpy as jnp
import numpy as np
from jax import lax
from jax.experimental import pallas as pl
from jax.experimental.pallas import tpu as pltpu
from jax.experimental.pallas import tpu_sc as plsc

KNN = 9
N = 10000
NPAD = 10240
C = 128
GRID_SIDE = 100
TBL_PAD = 384

RB = 512
NH = NPAD // 2
NBLKH = NH // RB
BIGF = 1e30
QSCALE = 524.0
QMAX = 64900
BIGKEY = 3.4e38

NW = 32
BWH = NH // NW
CH = 32
NCH = BWH // CH


def _xfp_body(xr_ref, rel_ref, tbl_ref, xfp_ref):
    tbl_iota = lax.broadcasted_iota(jnp.int32, (1, TBL_PAD), 1)
    onehot = jnp.where(rel_ref[...] == tbl_iota, 1.0, 0.0)
    ef = lax.dot_general(onehot, tbl_ref[...], (((1,), (0,)), ((), ())),
                         preferred_element_type=jnp.float32,
                         precision=lax.Precision.HIGHEST)
    xfp_ref[...] = xr_ref[...] + ef


def _xfp_call(xp, rel_col, tblp):
    XB = 512
    return pl.pallas_call(
        _xfp_body,
        grid=(NPAD // XB,),
        in_specs=[
            pl.BlockSpec((XB, C), lambda b: (b, 0)),
            pl.BlockSpec((XB, 1), lambda b: (b, 0)),
            pl.BlockSpec((TBL_PAD, C), lambda b: (0, 0)),
        ],
        out_specs=pl.BlockSpec((XB, C), lambda b: (b, 0)),
        out_shape=jax.ShapeDtypeStruct((NPAD, C), jnp.float32),
    )(xp, rel_col, tblp)


def _knn_body(xr_ref, x_ref, sqr_ref, sqc_ref, nbr_ref, *, hoff):
    b = pl.program_id(0)
    xr = xr_ref[...]
    x = x_ref[...]
    dot = lax.dot_general(xr, x, (((1,), (1,)), ((), ())),
                          preferred_element_type=jnp.float32)
    d = sqr_ref[...] - 2.0 * dot + sqc_ref[...]
    col = lax.broadcasted_iota(jnp.int32, (1, NPAD), 1)
    row_g = hoff + b * RB + lax.broadcasted_iota(jnp.int32, (RB, 1), 0)
    d = jnp.where(col == row_g, BIGF, d)
    b0 = jnp.min(d, axis=1, keepdims=True)
    q = jnp.minimum((d - b0) * QSCALE, float(QMAX)).astype(jnp.int32)
    key = lax.bitcast_convert_type(
        jnp.bitwise_or(jnp.left_shift(q, 14),
                       jnp.bitwise_or(col, 0x40000000)), jnp.float32)
    v = None
    for m in range(KNN):
        if m == 0:
            vm = jnp.min(key, axis=1, keepdims=True)
        else:
            vm = jnp.min(jnp.where(key > v, key, BIGKEY), axis=1, keepdims=True)
        ki = lax.bitcast_convert_type(vm, jnp.int32)
        nbr_ref[:, pl.ds(m, 1)] = jnp.bitwise_and(ki, 0x3FFF)
        v = vm
    nbr_ref[:, pl.ds(KNN, 16 - KNN)] = jnp.zeros((RB, 16 - KNN), jnp.int32)


def _knn_call(xp, sqr, sqc, half):
    hb = half * NBLKH
    return pl.pallas_call(
        functools.partial(_knn_body, hoff=half * NH),
        grid=(NBLKH,),
        in_specs=[
            pl.BlockSpec((RB, C), lambda b: (b + hb, 0)),
            pl.BlockSpec((NPAD, C), lambda b: (0, 0)),
            pl.BlockSpec((RB, 1), lambda b: (b + hb, 0)),
            pl.BlockSpec((1, NPAD), lambda b: (0, 0)),
        ],
        out_specs=pl.BlockSpec((RB, 16), lambda b: (b, 0)),
        out_shape=jax.ShapeDtypeStruct((NH, 16), jnp.int32),
    )(xp, xp, sqr, sqc)


def _sc_aggr_body(xfp_hbm, nbr3_hbm, out_hbm, idxw_v, own_v, g_v, res_v,
                  sem0, sem1, *, node_base):
    cid = lax.axis_index("c")
    sid = lax.axis_index("s")
    wid = sid * 2 + cid
    lbase = wid * BWH
    gbase = node_base + lbase
    pltpu.sync_copy(nbr3_hbm.at[wid], idxw_v)

    def fire(ci, sem, buf):
        off = ci * CH
        for m in range(KNN):
            pltpu.async_copy(
                xfp_hbm.at[idxw_v.at[m, pl.ds(off, CH)]], g_v.at[buf, m], sem)
        pltpu.async_copy(xfp_hbm.at[pl.ds(gbase + off, CH)], own_v.at[buf], sem)

    def drain(sem, buf):
        for m in range(KNN):
            pltpu.make_async_copy(
                xfp_hbm.at[pl.ds(0, CH)], g_v.at[buf, m], sem).wait()
        pltpu.make_async_copy(
            xfp_hbm.at[pl.ds(0, CH)], own_v.at[buf], sem).wait()

    def compute_store(ci, buf):
        def node(n, carry):
            for cc in range(C // 16):
                sl = pl.ds(cc * 16, 16)
                acc = g_v[buf, 0, n, sl]
                for m in range(1, KNN):
                    acc = jnp.maximum(acc, g_v[buf, m, n, sl])
                res_v[n, sl] = jnp.maximum(acc - own_v[buf, n, sl], 0.0)
            return carry

        lax.fori_loop(0, CH, node, 0)
        pltpu.sync_copy(res_v, out_hbm.at[pl.ds(lbase + ci * CH, CH)])

    fire(0, sem0, 0)

    def pair(p, carry):
        ci0 = 2 * p
        fire(ci0 + 1, sem1, 1)
        drain(sem0, 0)
        compute_store(ci0, 0)

        @pl.when(ci0 + 2 < NCH)
        def _():
            fire(ci0 + 2, sem0, 0)

        drain(sem1, 1)
        compute_store(ci0 + 1, 1)
        return carry

    lax.fori_loop(0, NCH // 2, pair, 0)
    if NCH % 2 == 1:
        ci = NCH - 1
        drain(sem0, 0)
        compute_store(ci, 0)


def _sc_aggr_call(xfp, nbr3, half):
    mesh = plsc.VectorSubcoreMesh(core_axis_name="c", subcore_axis_name="s")
    f = functools.partial(
        pl.kernel,
        mesh=mesh,
        out_type=jax.ShapeDtypeStruct((NH, C), jnp.float32),
        scratch_types=[
            pltpu.VMEM((KNN, BWH), jnp.int32),
            pltpu.VMEM((2, CH, C), jnp.float32),
            pltpu.VMEM((2, KNN, CH, C), jnp.float32),
            pltpu.VMEM((CH, C), jnp.float32),
            pltpu.SemaphoreType.DMA,
            pltpu.SemaphoreType.DMA,
        ],
    )(functools.partial(_sc_aggr_body, node_base=half * NH))
    return f(xfp, nbr3)


def _mm_body(w_ref, a_ref, o_ref):
    o_ref[...] = lax.dot_general(w_ref[...], a_ref[...],
                                 (((1,), (1,)), ((), ())),
                                 preferred_element_type=jnp.float32)


def _mm_call(W, aggr):
    MB = 512
    nb = aggr.shape[0] // MB
    return pl.pallas_call(
        _mm_body,
        grid=(nb,),
        in_specs=[
            pl.BlockSpec((C, C), lambda b: (0, 0)),
            pl.BlockSpec((MB, C), lambda b: (b, 0)),
        ],
        out_specs=pl.BlockSpec((C, MB), lambda b: (0, b)),
        out_shape=jax.ShapeDtypeStruct((C, aggr.shape[0]), jnp.float32),
    )(W, aggr)


_g = np.arange(GRID_SIDE)
_REL = (_g[:, None] - _g[None, :] + (GRID_SIDE - 1)).reshape(-1).astype(np.int32)
_REL_COL = np.zeros((NPAD, 1), np.int32)
_REL_COL[:N, 0] = _REL


def kernel(x, rel_pos_table, W):
    B, Cc, Nn = x.shape
    xf = jnp.transpose(x, (0, 2, 1)).reshape(Nn, Cc)
    xp = jnp.pad(xf, ((0, NPAD - N), (0, 0)))
    sq = jnp.sum(xf * xf, axis=1)
    sqp = jnp.pad(sq, (0, NPAD - N), constant_values=1e30)
    sqr = sqp.reshape(NPAD, 1)
    sqc = sqp.reshape(1, NPAD)
    tblp = jnp.pad(rel_pos_table, ((0, TBL_PAD - rel_pos_table.shape[0]), (0, 0)))
    rel_col = jnp.asarray(_REL_COL)

    xfp = _xfp_call(xp, rel_col, tblp)
    outs = []
    for h in range(2):
        nbr = _knn_call(xp, sqr, sqc, h)
        nbr3 = jnp.transpose(nbr[:, :KNN].reshape(NW, BWH, KNN), (0, 2, 1))
        aggr = _sc_aggr_call(xfp, nbr3, h)
        outs.append(_mm_call(W, aggr))
    out = jnp.concatenate(outs, axis=1)
    return out[:, :N].reshape(1, Cc, Nn)


# speedup vs baseline: 1.0067x; 1.0067x over previous
import functools

import jax
import jax.numpy as jnp
import numpy as np
from jax import lax
from jax.experimental import pallas as pl
from jax.experimental.pallas import tpu as pltpu
from jax.experimental.pallas import tpu_sc as plsc

KNN = 9
N = 10000
NPAD = 10240
C = 128
GRID_SIDE = 100
TBL_PAD = 384

RB = 256
NSPLIT = 4
NH = NPAD // NSPLIT
NBLKH = NH // RB
BIGF = 1e30
QSCALE = 524.0
QMAX = 64900
BIGKEY = 3.4e38

NW = 32
BWH = NH // NW
CH = 40
NCH = BWH // CH


def _xfp_body(xr_ref, rel_ref, tbl_ref, xfp_ref):
    tbl_iota = lax.broadcasted_iota(jnp.int32, (1, TBL_PAD), 1)
    onehot = jnp.where(rel_ref[...] == tbl_iota, 1.0, 0.0)
    ef = lax.dot_general(onehot, tbl_ref[...], (((1,), (0,)), ((), ())),
                         preferred_element_type=jnp.float32,
                         precision=lax.Precision.HIGHEST)
    xfp_ref[...] = xr_ref[...] + ef


def _xfp_call(xp, rel_col, tblp):
    XB = 512
    return pl.pallas_call(
        _xfp_body,
        grid=(NPAD // XB,),
        in_specs=[
            pl.BlockSpec((XB, C), lambda b: (b, 0)),
            pl.BlockSpec((XB, 1), lambda b: (b, 0)),
            pl.BlockSpec((TBL_PAD, C), lambda b: (0, 0)),
        ],
        out_specs=pl.BlockSpec((XB, C), lambda b: (b, 0)),
        out_shape=jax.ShapeDtypeStruct((NPAD, C), jnp.float32),
    )(xp, rel_col, tblp)


def _knn_body(xr_ref, x_ref, sqr_ref, sqc_ref, nbr_ref, *, hoff):
    b = pl.program_id(0)
    xr = xr_ref[...]
    x = x_ref[...]
    dot = lax.dot_general(xr, x, (((1,), (1,)), ((), ())),
                          preferred_element_type=jnp.float32)
    d = sqr_ref[...] - 2.0 * dot + sqc_ref[...]
    col = lax.broadcasted_iota(jnp.int32, (1, NPAD), 1)
    row_g = hoff + b * RB + lax.broadcasted_iota(jnp.int32, (RB, 1), 0)
    d = jnp.where(col == row_g, BIGF, d)
    b0 = jnp.min(d, axis=1, keepdims=True)
    q = jnp.minimum((d - b0) * QSCALE, float(QMAX)).astype(jnp.int32)
    key = lax.bitcast_convert_type(
        jnp.bitwise_or(jnp.left_shift(q, 14),
                       jnp.bitwise_or(col, 0x40000000)), jnp.float32)
    v = None
    for m in range(KNN):
        if m == 0:
            vm = jnp.min(key, axis=1, keepdims=True)
        else:
            vm = jnp.min(jnp.where(key > v, key, BIGKEY), axis=1, keepdims=True)
        ki = lax.bitcast_convert_type(vm, jnp.int32)
        nbr_ref[:, pl.ds(m, 1)] = jnp.bitwise_and(ki, 0x3FFF)
        v = vm
    nbr_ref[:, pl.ds(KNN, 16 - KNN)] = jnp.zeros((RB, 16 - KNN), jnp.int32)


def _knn_call(xp, sqr, sqc, half):
    hb = half * NBLKH
    return pl.pallas_call(
        functools.partial(_knn_body, hoff=half * NH),
        grid=(NBLKH,),
        in_specs=[
            pl.BlockSpec((RB, C), lambda b: (b + hb, 0)),
            pl.BlockSpec((NPAD, C), lambda b: (0, 0)),
            pl.BlockSpec((RB, 1), lambda b: (b + hb, 0)),
            pl.BlockSpec((1, NPAD), lambda b: (0, 0)),
        ],
        out_specs=pl.BlockSpec((RB, 16), lambda b: (b, 0)),
        out_shape=jax.ShapeDtypeStruct((NH, 16), jnp.int32),
    )(xp, xp, sqr, sqc)


def _sc_aggr_body(xfp_hbm, nbr3_hbm, out_hbm, idxw_v, own_v, g_v, res_v,
                  sem0, sem1, *, node_base):
    cid = lax.axis_index("c")
    sid = lax.axis_index("s")
    wid = sid * 2 + cid
    lbase = wid * BWH
    gbase = node_base + lbase
    pltpu.sync_copy(nbr3_hbm.at[wid], idxw_v)

    def fire(ci, sem, buf):
        off = ci * CH
        for m in range(KNN):
            pltpu.async_copy(
                xfp_hbm.at[idxw_v.at[m, pl.ds(off, CH)]], g_v.at[buf, m], sem)
        pltpu.async_copy(xfp_hbm.at[pl.ds(gbase + off, CH)], own_v.at[buf], sem)

    def drain(sem, buf):
        for m in range(KNN):
            pltpu.make_async_copy(
                xfp_hbm.at[pl.ds(0, CH)], g_v.at[buf, m], sem).wait()
        pltpu.make_async_copy(
            xfp_hbm.at[pl.ds(0, CH)], own_v.at[buf], sem).wait()

    def compute_store(ci, buf):
        def node(n, carry):
            for cc in range(C // 16):
                sl = pl.ds(cc * 16, 16)
                acc = g_v[buf, 0, n, sl]
                for m in range(1, KNN):
                    acc = jnp.maximum(acc, g_v[buf, m, n, sl])
                res_v[n, sl] = jnp.maximum(acc - own_v[buf, n, sl], 0.0)
            return carry

        lax.fori_loop(0, CH, node, 0)
        pltpu.sync_copy(res_v, out_hbm.at[pl.ds(lbase + ci * CH, CH)])

    fire(0, sem0, 0)

    def pair(p, carry):
        ci0 = 2 * p
        fire(ci0 + 1, sem1, 1)
        drain(sem0, 0)
        compute_store(ci0, 0)

        @pl.when(ci0 + 2 < NCH)
        def _():
            fire(ci0 + 2, sem0, 0)

        drain(sem1, 1)
        compute_store(ci0 + 1, 1)
        return carry

    lax.fori_loop(0, NCH // 2, pair, 0)
    if NCH % 2 == 1:
        ci = NCH - 1
        drain(sem0, 0)
        compute_store(ci, 0)


def _sc_aggr_call(xfp, nbr3, half):
    mesh = plsc.VectorSubcoreMesh(core_axis_name="c", subcore_axis_name="s")
    f = functools.partial(
        pl.kernel,
        mesh=mesh,
        out_type=jax.ShapeDtypeStruct((NH, C), jnp.float32),
        scratch_types=[
            pltpu.VMEM((KNN, BWH), jnp.int32),
            pltpu.VMEM((2, CH, C), jnp.float32),
            pltpu.VMEM((2, KNN, CH, C), jnp.float32),
            pltpu.VMEM((CH, C), jnp.float32),
            pltpu.SemaphoreType.DMA,
            pltpu.SemaphoreType.DMA,
        ],
    )(functools.partial(_sc_aggr_body, node_base=half * NH))
    return f(xfp, nbr3)


def _mm_body(w_ref, a_ref, o_ref):
    o_ref[...] = lax.dot_general(w_ref[...], a_ref[...],
                                 (((1,), (1,)), ((), ())),
                                 preferred_element_type=jnp.float32)


def _mm_call(W, aggr):
    MB = 512
    nb = aggr.shape[0] // MB
    return pl.pallas_call(
        _mm_body,
        grid=(nb,),
        in_specs=[
            pl.BlockSpec((C, C), lambda b: (0, 0)),
            pl.BlockSpec((MB, C), lambda b: (b, 0)),
        ],
        out_specs=pl.BlockSpec((C, MB), lambda b: (0, b)),
        out_shape=jax.ShapeDtypeStruct((C, aggr.shape[0]), jnp.float32),
    )(W, aggr)


_g = np.arange(GRID_SIDE)
_REL = (_g[:, None] - _g[None, :] + (GRID_SIDE - 1)).reshape(-1).astype(np.int32)
_REL_COL = np.zeros((NPAD, 1), np.int32)
_REL_COL[:N, 0] = _REL


def kernel(x, rel_pos_table, W):
    B, Cc, Nn = x.shape
    xf = jnp.transpose(x, (0, 2, 1)).reshape(Nn, Cc)
    xp = jnp.pad(xf, ((0, NPAD - N), (0, 0)))
    sq = jnp.sum(xf * xf, axis=1)
    sqp = jnp.pad(sq, (0, NPAD - N), constant_values=1e30)
    sqr = sqp.reshape(NPAD, 1)
    sqc = sqp.reshape(1, NPAD)
    tblp = jnp.pad(rel_pos_table, ((0, TBL_PAD - rel_pos_table.shape[0]), (0, 0)))
    rel_col = jnp.asarray(_REL_COL)

    xfp = _xfp_call(xp, rel_col, tblp)
    outs = []
    for h in range(NSPLIT):
        nbr = _knn_call(xp, sqr, sqc, h)
        nbr3 = jnp.transpose(nbr[:, :KNN].reshape(NW, BWH, KNN), (0, 2, 1))
        aggr = _sc_aggr_call(xfp, nbr3, h)
        outs.append(_mm_call(W, aggr))
    out = jnp.concatenate(outs, axis=1)
    return out[:, :N].reshape(1, Cc, Nn)


# speedup vs baseline: 1.0310x; 1.0242x over previous
import functools

import jax
import jax.numpy as jnp
import numpy as np
from jax import lax
from jax.experimental import pallas as pl
from jax.experimental.pallas import tpu as pltpu
from jax.experimental.pallas import tpu_sc as plsc

KNN = 9
N = 10000
NPAD = 10240
C = 128
GRID_SIDE = 100
TBL_PAD = 384

RB = 256
NSPLIT = 2
NH = NPAD // NSPLIT
NBLKH = NH // RB
BIGF = 1e30
QSCALE = 524.0
QMAX = 64900
BIGKEY = 3.4e38

NW = 32
BWH = NH // NW
CH = 32
NCH = BWH // CH


def _xfp_body(xr_ref, rel_ref, tbl_ref, xfp_ref):
    tbl_iota = lax.broadcasted_iota(jnp.int32, (1, TBL_PAD), 1)
    onehot = jnp.where(rel_ref[...] == tbl_iota, 1.0, 0.0)
    ef = lax.dot_general(onehot, tbl_ref[...], (((1,), (0,)), ((), ())),
                         preferred_element_type=jnp.float32,
                         precision=lax.Precision.HIGHEST)
    xfp_ref[...] = xr_ref[...] + ef


def _xfp_call(xp, rel_col, tblp):
    XB = 512
    return pl.pallas_call(
        _xfp_body,
        grid=(NPAD // XB,),
        in_specs=[
            pl.BlockSpec((XB, C), lambda b: (b, 0)),
            pl.BlockSpec((XB, 1), lambda b: (b, 0)),
            pl.BlockSpec((TBL_PAD, C), lambda b: (0, 0)),
        ],
        out_specs=pl.BlockSpec((XB, C), lambda b: (b, 0)),
        out_shape=jax.ShapeDtypeStruct((NPAD, C), jnp.float32),
    )(xp, rel_col, tblp)


def _knn_body(xr_ref, x_ref, sqr_ref, sqc_ref, nbr_ref, *, hoff):
    b = pl.program_id(0)
    xr = xr_ref[...]
    x = x_ref[...]
    dot = lax.dot_general(xr, x, (((1,), (1,)), ((), ())),
                          preferred_element_type=jnp.float32)
    d = sqr_ref[...] - 2.0 * dot + sqc_ref[...]
    col = lax.broadcasted_iota(jnp.int32, (1, NPAD), 1)
    row_g = hoff + b * RB + lax.broadcasted_iota(jnp.int32, (RB, 1), 0)
    d = jnp.where(col == row_g, BIGF, d)
    b0 = jnp.min(d, axis=1, keepdims=True)
    q = jnp.minimum((d - b0) * QSCALE, float(QMAX)).astype(jnp.int32)
    key = lax.bitcast_convert_type(
        jnp.bitwise_or(jnp.left_shift(q, 14),
                       jnp.bitwise_or(col, 0x40000000)), jnp.float32)
    v = None
    for m in range(KNN):
        if m == 0:
            vm = jnp.min(key, axis=1, keepdims=True)
        else:
            vm = jnp.min(jnp.where(key > v, key, BIGKEY), axis=1, keepdims=True)
        ki = lax.bitcast_convert_type(vm, jnp.int32)
        nbr_ref[:, pl.ds(m, 1)] = jnp.bitwise_and(ki, 0x3FFF)
        v = vm
    nbr_ref[:, pl.ds(KNN, 16 - KNN)] = jnp.zeros((RB, 16 - KNN), jnp.int32)


def _knn_call(xp, sqr, sqc, half):
    hb = half * NBLKH
    return pl.pallas_call(
        functools.partial(_knn_body, hoff=half * NH),
        grid=(NBLKH,),
        in_specs=[
            pl.BlockSpec((RB, C), lambda b: (b + hb, 0)),
            pl.BlockSpec((NPAD, C), lambda b: (0, 0)),
            pl.BlockSpec((RB, 1), lambda b: (b + hb, 0)),
            pl.BlockSpec((1, NPAD), lambda b: (0, 0)),
        ],
        out_specs=pl.BlockSpec((RB, 16), lambda b: (b, 0)),
        out_shape=jax.ShapeDtypeStruct((NH, 16), jnp.int32),
    )(xp, xp, sqr, sqc)


def _sc_aggr_body(xfp_hbm, nbr3_hbm, out_hbm, idxw_v, own_v, g_v, res_v,
                  sem0, sem1, *, node_base):
    cid = lax.axis_index("c")
    sid = lax.axis_index("s")
    wid = sid * 2 + cid
    lbase = wid * BWH
    gbase = node_base + lbase
    pltpu.sync_copy(nbr3_hbm.at[wid], idxw_v)

    def fire(ci, sem, buf):
        off = ci * CH
        for m in range(KNN):
            pltpu.async_copy(
                xfp_hbm.at[idxw_v.at[m, pl.ds(off, CH)]], g_v.at[buf, m], sem)
        pltpu.async_copy(xfp_hbm.at[pl.ds(gbase + off, CH)], own_v.at[buf], sem)

    def drain(sem, buf):
        for m in range(KNN):
            pltpu.make_async_copy(
                xfp_hbm.at[pl.ds(0, CH)], g_v.at[buf, m], sem).wait()
        pltpu.make_async_copy(
            xfp_hbm.at[pl.ds(0, CH)], own_v.at[buf], sem).wait()

    def compute_store(ci, buf):
        def node(n, carry):
            for cc in range(C // 16):
                sl = pl.ds(cc * 16, 16)
                acc = g_v[buf, 0, n, sl]
                for m in range(1, KNN):
                    acc = jnp.maximum(acc, g_v[buf, m, n, sl])
                res_v[n, sl] = jnp.maximum(acc - own_v[buf, n, sl], 0.0)
            return carry

        lax.fori_loop(0, CH, node, 0)
        pltpu.sync_copy(res_v, out_hbm.at[pl.ds(lbase + ci * CH, CH)])

    fire(0, sem0, 0)

    def pair(p, carry):
        ci0 = 2 * p
        fire(ci0 + 1, sem1, 1)
        drain(sem0, 0)
        compute_store(ci0, 0)

        @pl.when(ci0 + 2 < NCH)
        def _():
            fire(ci0 + 2, sem0, 0)

        drain(sem1, 1)
        compute_store(ci0 + 1, 1)
        return carry

    lax.fori_loop(0, NCH // 2, pair, 0)
    if NCH % 2 == 1:
        ci = NCH - 1
        drain(sem0, 0)
        compute_store(ci, 0)


def _sc_aggr_call(xfp, nbr3, half):
    mesh = plsc.VectorSubcoreMesh(core_axis_name="c", subcore_axis_name="s")
    f = functools.partial(
        pl.kernel,
        mesh=mesh,
        out_type=jax.ShapeDtypeStruct((NH, C), jnp.float32),
        scratch_types=[
            pltpu.VMEM((KNN, BWH), jnp.int32),
            pltpu.VMEM((2, CH, C), jnp.float32),
            pltpu.VMEM((2, KNN, CH, C), jnp.float32),
            pltpu.VMEM((CH, C), jnp.float32),
            pltpu.SemaphoreType.DMA,
            pltpu.SemaphoreType.DMA,
        ],
    )(functools.partial(_sc_aggr_body, node_base=half * NH))
    return f(xfp, nbr3)


def _mm_body(w_ref, a_ref, o_ref):
    o_ref[...] = lax.dot_general(w_ref[...], a_ref[...],
                                 (((1,), (1,)), ((), ())),
                                 preferred_element_type=jnp.float32)


def _mm_call(W, aggr):
    MB = 512
    nb = aggr.shape[0] // MB
    return pl.pallas_call(
        _mm_body,
        grid=(nb,),
        in_specs=[
            pl.BlockSpec((C, C), lambda b: (0, 0)),
            pl.BlockSpec((MB, C), lambda b: (b, 0)),
        ],
        out_specs=pl.BlockSpec((C, MB), lambda b: (0, b)),
        out_shape=jax.ShapeDtypeStruct((C, aggr.shape[0]), jnp.float32),
    )(W, aggr)


_g = np.arange(GRID_SIDE)
_REL = (_g[:, None] - _g[None, :] + (GRID_SIDE - 1)).reshape(-1).astype(np.int32)
_REL_COL = np.zeros((NPAD, 1), np.int32)
_REL_COL[:N, 0] = _REL


def kernel(x, rel_pos_table, W):
    B, Cc, Nn = x.shape
    xf = jnp.transpose(x, (0, 2, 1)).reshape(Nn, Cc)
    xp = jnp.pad(xf, ((0, NPAD - N), (0, 0)))
    sq = jnp.sum(xf * xf, axis=1)
    sqp = jnp.pad(sq, (0, NPAD - N), constant_values=1e30)
    sqr = sqp.reshape(NPAD, 1)
    sqc = sqp.reshape(1, NPAD)
    tblp = jnp.pad(rel_pos_table, ((0, TBL_PAD - rel_pos_table.shape[0]), (0, 0)))
    rel_col = jnp.asarray(_REL_COL)

    xfp = _xfp_call(xp, rel_col, tblp)
    outs = []
    for h in range(NSPLIT):
        nbr = _knn_call(xp, sqr, sqc, h)
        nbr3 = jnp.transpose(nbr[:, :KNN].reshape(NW, BWH, KNN), (0, 2, 1))
        aggr = _sc_aggr_call(xfp, nbr3, h)
        outs.append(_mm_call(W, aggr))
    out = jnp.concatenate(outs, axis=1)
    return out[:, :N].reshape(1, Cc, Nn)
